# baseline (speedup 1.0000x reference)
import functools

import jax
import jax.numpy as jnp
from jax import lax
from jax.experimental import pallas as pl
from jax.experimental.pallas import tpu as pltpu
from jax.experimental.pallas import tpu_sc as plsc

N = 10000
E = 320000
D = 128
DOUT = 4

NC = 2
NS = 16
NW = NC * NS
CHUNK = 128
NCHUNK = E // CHUNK
BASE_CH = NCHUNK // NW
EXTRA_W = 2
NRING = 4
NGROUP = (BASE_CH + 2 + 3) // 4
RPT = 624
RPT_LAST = N - (NS - 1) * RPT
DEGW = 16

_MESH = plsc.VectorSubcoreMesh(core_axis_name="c", subcore_axis_name="s")


def _fill(ref, rows, width, val):
    vec = jnp.full((16,), val, jnp.float32)

    def body(i, carry):
        for k in range(width // 16):
            ref[i, pl.ds(k * 16, 16)] = vec
        return carry

    lax.fori_loop(0, rows, body, 0)


def _zero_slab(acc, zbuf, row0, nrows):
    full = nrows // CHUNK
    for t in range(full):
        pltpu.sync_copy(zbuf, acc.at[pl.ds(row0 + t * CHUNK, CHUNK)])
    rem = nrows - full * CHUNK
    if rem:
        pltpu.sync_copy(zbuf.at[pl.ds(0, rem)],
                        acc.at[pl.ds(row0 + full * CHUNK, rem)])


def _per_tile_rows(sid, fn):
    @pl.when(sid < NS - 1)
    def _():
        fn(sid * RPT, RPT)

    @pl.when(sid == NS - 1)
    def _():
        fn(sid * RPT, RPT_LAST)


def _worker_span(w):
    start = w * BASE_CH + 2 * jnp.minimum(w, EXTRA_W)
    cnt = BASE_CH + 2 * (w < EXTRA_W).astype(jnp.int32)
    return start, cnt


@functools.partial(
    pl.kernel,
    out_type=jax.ShapeDtypeStruct((NC * N, DEGW), jnp.float32),
    mesh=_MESH,
    scratch_types=[
        pltpu.VMEM_SHARED((N, DEGW), jnp.float32),
        pltpu.VMEM((CHUNK,), jnp.int32),
        pltpu.VMEM((CHUNK,), jnp.int32),
        pltpu.VMEM((CHUNK,), jnp.int32),
        pltpu.VMEM((CHUNK,), jnp.int32),
        pltpu.VMEM((CHUNK,), jnp.int32),
        pltpu.VMEM((CHUNK,), jnp.int32),
        pltpu.VMEM((CHUNK, DEGW), jnp.float32),
        pltpu.VMEM((CHUNK, DEGW), jnp.float32),
        pltpu.SemaphoreType.DMA,
        pltpu.SemaphoreType.DMA,
        pltpu.SemaphoreType.DMA,
        pltpu.SemaphoreType.DMA,
        pltpu.SemaphoreType.DMA,
        pltpu.SemaphoreType.DMA,
    ],
)
def _deg(dst_hbm, out_hbm, acc, dx0, dx1, dx2, dx3, dx4, dx5,
         ones_b, zero_b, i0, i1, i2, i3, i4, i5):
    cid = lax.axis_index("c")
    sid = lax.axis_index("s")
    w = cid * NS + sid
    start, cnt = _worker_span(w)
    didx = (dx0, dx1, dx2, dx3, dx4, dx5)
    isem = (i0, i1, i2, i3, i4, i5)

    _fill(ones_b, CHUNK, DEGW, 1.0)
    _fill(zero_b, CHUNK, DEGW, 0.0)
    _per_tile_rows(sid, lambda r0, nr: _zero_slab(acc, zero_b, r0, nr))
    plsc.subcore_barrier()

    def block(chunk0, n):
        hd = []
        for t in range(n):
            hd.append(pltpu.async_copy(
                dst_hbm.at[pl.ds((chunk0 + t) * CHUNK, CHUNK)],
                didx[t], isem[t]))
        for t in range(n):
            hd[t].wait()
        hc = []
        for t in range(n):
            hc.append(pltpu.async_copy(ones_b, acc.at[didx[t]], isem[t],
                                       add=True))
        for t in range(n):
            hc[t].wait()

    def body(g, carry):
        block(start + 6 * g, 6)
        return carry

    nfull = cnt // 6
    lax.fori_loop(0, nfull, body, 0)

    @pl.when(cnt - 6 * nfull == 2)
    def _():
        block(start + 6 * nfull, 2)
    plsc.subcore_barrier()
    _per_tile_rows(sid, lambda r0, nr: pltpu.sync_copy(
        acc.at[pl.ds(r0, nr)], out_hbm.at[pl.ds(cid * N + r0, nr)]))


AGG_U = 3


@functools.partial(
    pl.kernel,
    out_type=jax.ShapeDtypeStruct((NC * N, D), jnp.float32),
    mesh=_MESH,
    scratch_types=[
        pltpu.VMEM_SHARED((N, D), jnp.float32),
        pltpu.VMEM((AGG_U * CHUNK,), jnp.int32),
        pltpu.VMEM((CHUNK,), jnp.int32),
        pltpu.VMEM((CHUNK,), jnp.int32),
        pltpu.VMEM((CHUNK,), jnp.int32),
        pltpu.VMEM((CHUNK, D), jnp.float32),
        pltpu.VMEM((CHUNK, D), jnp.float32),
        pltpu.VMEM((CHUNK, D), jnp.float32),
        pltpu.SemaphoreType.DMA,
        pltpu.SemaphoreType.DMA,
        pltpu.SemaphoreType.DMA,
        pltpu.SemaphoreType.DMA,
        pltpu.SemaphoreType.DMA,
        pltpu.SemaphoreType.DMA,
        pltpu.SemaphoreType.DMA,
    ],
)
def _agg(y_hbm, src_hbm, dst_hbm, out_hbm, acc,
         sxa, dx0, dx1, dx2, rb0, rb1, rb2,
         is0, id0, id1, id2, ig0, ig1, ig2):
    cid = lax.axis_index("c")
    sid = lax.axis_index("s")
    w = cid * NS + sid
    start, cnt = _worker_span(w)
    didx = (dx0, dx1, dx2)
    rows = (rb0, rb1, rb2)
    isem_d = (id0, id1, id2)
    gsem = (ig0, ig1, ig2)

    _fill(rb0, CHUNK, D, 0.0)
    _per_tile_rows(sid, lambda a, n: _zero_slab(acc, rb0, a, n))
    plsc.subcore_barrier()

    def block(chunk0, n):
        hd, hg = [], []
        hs = pltpu.async_copy(src_hbm.at[pl.ds(chunk0 * CHUNK, n * CHUNK)],
                              sxa.at[pl.ds(0, n * CHUNK)], is0)
        for t in range(n):
            off = (chunk0 + t) * CHUNK
            hd.append(pltpu.async_copy(dst_hbm.at[pl.ds(off, CHUNK)],
                                       didx[t], isem_d[t]))
        hs.wait()
        for t in range(n):
            hg.append(pltpu.async_copy(
                y_hbm.at[sxa.at[pl.ds(t * CHUNK, CHUNK)]], rows[t], gsem[t]))
        for t in range(n):
            hg[t].wait()
        for t in range(n):
            hd[t].wait()
        hc = []
        for t in range(n):
            hc.append(pltpu.async_copy(rows[t], acc.at[didx[t]], gsem[t],
                                       add=True))
        for t in range(n):
            hc[t].wait()

    def body(g, carry):
        block(start + AGG_U * g, AGG_U)
        return carry

    nfull = cnt // AGG_U
    lax.fori_loop(0, nfull, body, 0)

    @pl.when(cnt - AGG_U * nfull == 2)
    def _():
        block(start + AGG_U * nfull, 2)

    plsc.subcore_barrier()
    _per_tile_rows(sid, lambda a, n: pltpu.sync_copy(
        acc.at[pl.ds(a, n)], out_hbm.at[pl.ds(cid * N + a, n)]))


BLK = 2000


def _dinv_of(degt_ref):
    deg = degt_ref[0, :, 0] + degt_ref[1, :, 0] + 1.0
    return lax.rsqrt(deg)[:, None]


def _mmscale_body(x_ref, w_ref, degt_ref, y_ref):
    y_ref[...] = jnp.dot(x_ref[...], w_ref[...],
                         preferred_element_type=jnp.float32) * _dinv_of(degt_ref)


def _mmscale(x, w, degt):
    return pl.pallas_call(
        _mmscale_body,
        grid=(N // BLK,),
        in_specs=[pl.BlockSpec((BLK, D), lambda i: (i, 0)),
                  pl.BlockSpec((D, D), lambda i: (0, 0)),
                  pl.BlockSpec((NC, BLK, DEGW), lambda i: (0, i, 0))],
        out_specs=pl.BlockSpec((BLK, D), lambda i: (i, 0)),
        out_shape=jax.ShapeDtypeStruct((N, D), jnp.float32),
    )(x, w, degt)


def _mid_body(p_ref, y_ref, degt_ref, b_ref, w_ref, o_ref):
    dinv = _dinv_of(degt_ref)
    h = (p_ref[0] + p_ref[1] + y_ref[...]) * dinv + b_ref[...]
    h = jnp.maximum(h, 0.0)
    o_ref[...] = jnp.dot(h, w_ref[...],
                         preferred_element_type=jnp.float32) * dinv


def _mid(p, y, degt, b, w):
    return pl.pallas_call(
        _mid_body,
        grid=(N // BLK,),
        in_specs=[pl.BlockSpec((NC, BLK, D), lambda i: (0, i, 0)),
                  pl.BlockSpec((BLK, D), lambda i: (i, 0)),
                  pl.BlockSpec((NC, BLK, DEGW), lambda i: (0, i, 0)),
                  pl.BlockSpec((1, D), lambda i: (0, 0)),
                  pl.BlockSpec((D, D), lambda i: (0, 0))],
        out_specs=pl.BlockSpec((BLK, D), lambda i: (i, 0)),
        out_shape=jax.ShapeDtypeStruct((N, D), jnp.float32),
    )(p, y, degt, b, w)


def _fin_body(p_ref, y_ref, degt_ref, b_ref, wl_ref, bl_ref, o_ref):
    dinv = _dinv_of(degt_ref)
    h = (p_ref[0] + p_ref[1] + y_ref[...]) * dinv + b_ref[...]
    o_ref[...] = jnp.dot(h, wl_ref[...],
                         preferred_element_type=jnp.float32) + bl_ref[...]


def _fin(p, y, degt, b, wl, bl):
    return pl.pallas_call(
        _fin_body,
        grid=(N // BLK,),
        in_specs=[pl.BlockSpec((NC, BLK, D), lambda i: (0, i, 0)),
                  pl.BlockSpec((BLK, D), lambda i: (i, 0)),
                  pl.BlockSpec((NC, BLK, DEGW), lambda i: (0, i, 0)),
                  pl.BlockSpec((1, D), lambda i: (0, 0)),
                  pl.BlockSpec((D, DOUT), lambda i: (0, 0)),
                  pl.BlockSpec((1, DOUT), lambda i: (0, 0))],
        out_specs=pl.BlockSpec((BLK, DOUT), lambda i: (i, 0)),
        out_shape=jax.ShapeDtypeStruct((N, DOUT), jnp.float32),
    )(p, y, degt, b, wl, bl)


def kernel(x, edge_index, PQVA_mask, target_vector, W1, b1, W2, b2, Wl, bl):
    src = edge_index[0]
    dst = edge_index[1]
    degt = _deg(dst).reshape(NC, N, DEGW)
    y1 = _mmscale(x, W1, degt)
    p1 = _agg(y1, src, dst).reshape(NC, N, D)
    y2 = _mid(p1, y1, degt, b1.reshape(1, D), W2)
    p2 = _agg(y2, src, dst).reshape(NC, N, D)
    return _fin(p2, y2, degt, b2.reshape(1, D), Wl, bl.reshape(1, DOUT))


# speedup vs baseline: 1.0011x; 1.0011x over previous
import functools

import jax
import jax.numpy as jnp
from jax import lax
from jax.experimental import pallas as pl
from jax.experimental.pallas import tpu as pltpu
from jax.experimental.pallas import tpu_sc as plsc

N = 10000
E = 320000
D = 128
DOUT = 4

NC = 2
NS = 16
NW = NC * NS
CHUNK = 128
NCHUNK = E // CHUNK
BASE_CH = NCHUNK // NW
EXTRA_W = 2
NRING = 4
NGROUP = (BASE_CH + 2 + 3) // 4
RPT = 624
RPT_LAST = N - (NS - 1) * RPT
DEGW = 16

_MESH = plsc.VectorSubcoreMesh(core_axis_name="c", subcore_axis_name="s")


def _fill(ref, rows, width, val):
    vec = jnp.full((16,), val, jnp.float32)

    def body(i, carry):
        for k in range(width // 16):
            ref[i, pl.ds(k * 16, 16)] = vec
        return carry

    lax.fori_loop(0, rows, body, 0)


def _zero_slab(acc, zbuf, row0, nrows):
    full = nrows // CHUNK
    for t in range(full):
        pltpu.sync_copy(zbuf, acc.at[pl.ds(row0 + t * CHUNK, CHUNK)])
    rem = nrows - full * CHUNK
    if rem:
        pltpu.sync_copy(zbuf.at[pl.ds(0, rem)],
                        acc.at[pl.ds(row0 + full * CHUNK, rem)])


def _per_tile_rows(sid, fn):
    @pl.when(sid < NS - 1)
    def _():
        fn(sid * RPT, RPT)

    @pl.when(sid == NS - 1)
    def _():
        fn(sid * RPT, RPT_LAST)


def _worker_span(w):
    start = w * BASE_CH + 2 * jnp.minimum(w, EXTRA_W)
    cnt = BASE_CH + 2 * (w < EXTRA_W).astype(jnp.int32)
    return start, cnt


@functools.partial(
    pl.kernel,
    out_type=jax.ShapeDtypeStruct((NC * N, DEGW), jnp.float32),
    mesh=_MESH,
    scratch_types=[
        pltpu.VMEM_SHARED((N, DEGW), jnp.float32),
        pltpu.VMEM((CHUNK,), jnp.int32),
        pltpu.VMEM((CHUNK,), jnp.int32),
        pltpu.VMEM((CHUNK,), jnp.int32),
        pltpu.VMEM((CHUNK,), jnp.int32),
        pltpu.VMEM((CHUNK,), jnp.int32),
        pltpu.VMEM((CHUNK,), jnp.int32),
        pltpu.VMEM((CHUNK, DEGW), jnp.float32),
        pltpu.VMEM((CHUNK, DEGW), jnp.float32),
        pltpu.SemaphoreType.DMA,
        pltpu.SemaphoreType.DMA,
        pltpu.SemaphoreType.DMA,
        pltpu.SemaphoreType.DMA,
        pltpu.SemaphoreType.DMA,
        pltpu.SemaphoreType.DMA,
    ],
)
def _deg(dst_hbm, out_hbm, acc, dx0, dx1, dx2, dx3, dx4, dx5,
         ones_b, zero_b, i0, i1, i2, i3, i4, i5):
    cid = lax.axis_index("c")
    sid = lax.axis_index("s")
    w = cid * NS + sid
    start, cnt = _worker_span(w)
    didx = (dx0, dx1, dx2, dx3, dx4, dx5)
    isem = (i0, i1, i2, i3, i4, i5)

    _fill(ones_b, CHUNK, DEGW, 1.0)
    _fill(zero_b, CHUNK, DEGW, 0.0)
    _per_tile_rows(sid, lambda r0, nr: _zero_slab(acc, zero_b, r0, nr))
    plsc.subcore_barrier()

    def block(chunk0, n):
        hd = []
        for t in range(n):
            hd.append(pltpu.async_copy(
                dst_hbm.at[pl.ds((chunk0 + t) * CHUNK, CHUNK)],
                didx[t], isem[t]))
        for t in range(n):
            hd[t].wait()
        hc = []
        for t in range(n):
            hc.append(pltpu.async_copy(ones_b, acc.at[didx[t]], isem[t],
                                       add=True))
        for t in range(n):
            hc[t].wait()

    def body(g, carry):
        block(start + 6 * g, 6)
        return carry

    nfull = cnt // 6
    lax.fori_loop(0, nfull, body, 0)

    @pl.when(cnt - 6 * nfull == 2)
    def _():
        block(start + 6 * nfull, 2)
    plsc.subcore_barrier()
    _per_tile_rows(sid, lambda r0, nr: pltpu.sync_copy(
        acc.at[pl.ds(r0, nr)], out_hbm.at[pl.ds(cid * N + r0, nr)]))


AGG_U = 3


@functools.partial(
    pl.kernel,
    out_type=jax.ShapeDtypeStruct((NC * N, D), jnp.float32),
    mesh=_MESH,
    scratch_types=[
        pltpu.VMEM_SHARED((N, D), jnp.float32),
        pltpu.VMEM((AGG_U * CHUNK,), jnp.int32),
        pltpu.VMEM((CHUNK,), jnp.int32),
        pltpu.VMEM((CHUNK,), jnp.int32),
        pltpu.VMEM((CHUNK,), jnp.int32),
        pltpu.VMEM((CHUNK, D), jnp.float32),
        pltpu.VMEM((CHUNK, D), jnp.float32),
        pltpu.VMEM((CHUNK, D), jnp.float32),
        pltpu.SemaphoreType.DMA,
        pltpu.SemaphoreType.DMA,
        pltpu.SemaphoreType.DMA,
        pltpu.SemaphoreType.DMA,
        pltpu.SemaphoreType.DMA,
        pltpu.SemaphoreType.DMA,
        pltpu.SemaphoreType.DMA,
    ],
)
def _agg(y_hbm, src_hbm, dst_hbm, out_hbm, acc,
         sxa, dx0, dx1, dx2, rb0, rb1, rb2,
         is0, id0, id1, id2, ig0, ig1, ig2):
    cid = lax.axis_index("c")
    sid = lax.axis_index("s")
    w = cid * NS + sid
    start, cnt = _worker_span(w)
    didx = (dx0, dx1, dx2)
    rows = (rb0, rb1, rb2)
    isem_d = (id0, id1, id2)
    gsem = (ig0, ig1, ig2)

    _fill(rb0, CHUNK, D, 0.0)
    _per_tile_rows(sid, lambda a, n: _zero_slab(acc, rb0, a, n))
    plsc.subcore_barrier()

    def block(chunk0, n):
        hd, hg = [], []
        hs = pltpu.async_copy(src_hbm.at[pl.ds(chunk0 * CHUNK, n * CHUNK)],
                              sxa.at[pl.ds(0, n * CHUNK)], is0)
        for t in range(n):
            off = (chunk0 + t) * CHUNK
            hd.append(pltpu.async_copy(dst_hbm.at[pl.ds(off, CHUNK)],
                                       didx[t], isem_d[t]))
        hs.wait()
        for t in range(n):
            hg.append(pltpu.async_copy(
                y_hbm.at[sxa.at[pl.ds(t * CHUNK, CHUNK)]], rows[t], gsem[t]))
        for t in range(n):
            hg[t].wait()
        for t in range(n):
            hd[t].wait()
        hc = []
        for t in range(n):
            hc.append(pltpu.async_copy(rows[t], acc.at[didx[t]], gsem[t],
                                       add=True))
        for t in range(n):
            hc[t].wait()

    def body(g, carry):
        block(start + AGG_U * g, AGG_U)
        return carry

    nfull = cnt // AGG_U
    lax.fori_loop(0, nfull, body, 0)

    @pl.when(cnt - AGG_U * nfull == 2)
    def _():
        block(start + AGG_U * nfull, 2)

    plsc.subcore_barrier()
    _per_tile_rows(sid, lambda a, n: pltpu.sync_copy(
        acc.at[pl.ds(a, n)], out_hbm.at[pl.ds(cid * N + a, n)]))


BLK = 2000


def _dinv_of(degt_ref):
    deg = degt_ref[0, :, 0] + degt_ref[1, :, 0] + 1.0
    return lax.rsqrt(deg)[:, None]


def _mm_body(x_ref, w_ref, o_ref):
    o_ref[...] = jnp.dot(x_ref[...], w_ref[...],
                         preferred_element_type=jnp.float32)


def _mm(x, w):
    m, k = x.shape
    n = w.shape[1]
    return pl.pallas_call(
        _mm_body,
        grid=(m // BLK,),
        in_specs=[pl.BlockSpec((BLK, k), lambda i: (i, 0)),
                  pl.BlockSpec((k, n), lambda i: (0, 0))],
        out_specs=pl.BlockSpec((BLK, n), lambda i: (i, 0)),
        out_shape=jax.ShapeDtypeStruct((m, n), jnp.float32),
    )(x, w)


def _scale_body(xw_ref, degt_ref, y_ref):
    y_ref[...] = xw_ref[...] * _dinv_of(degt_ref)


def _scale(xw, degt):
    return pl.pallas_call(
        _scale_body,
        grid=(N // BLK,),
        in_specs=[pl.BlockSpec((BLK, D), lambda i: (i, 0)),
                  pl.BlockSpec((NC, BLK, DEGW), lambda i: (0, i, 0))],
        out_specs=pl.BlockSpec((BLK, D), lambda i: (i, 0)),
        out_shape=jax.ShapeDtypeStruct((N, D), jnp.float32),
    )(xw, degt)


def _mid_body(p_ref, y_ref, degt_ref, b_ref, w_ref, o_ref):
    dinv = _dinv_of(degt_ref)
    h = (p_ref[0] + p_ref[1] + y_ref[...]) * dinv + b_ref[...]
    h = jnp.maximum(h, 0.0)
    o_ref[...] = jnp.dot(h, w_ref[...],
                         preferred_element_type=jnp.float32) * dinv


def _mid(p, y, degt, b, w):
    return pl.pallas_call(
        _mid_body,
        grid=(N // BLK,),
        in_specs=[pl.BlockSpec((NC, BLK, D), lambda i: (0, i, 0)),
                  pl.BlockSpec((BLK, D), lambda i: (i, 0)),
                  pl.BlockSpec((NC, BLK, DEGW), lambda i: (0, i, 0)),
                  pl.BlockSpec((1, D), lambda i: (0, 0)),
                  pl.BlockSpec((D, D), lambda i: (0, 0))],
        out_specs=pl.BlockSpec((BLK, D), lambda i: (i, 0)),
        out_shape=jax.ShapeDtypeStruct((N, D), jnp.float32),
    )(p, y, degt, b, w)


def _fin_body(p_ref, y_ref, degt_ref, b_ref, wl_ref, bl_ref, o_ref):
    dinv = _dinv_of(degt_ref)
    h = (p_ref[0] + p_ref[1] + y_ref[...]) * dinv + b_ref[...]
    o_ref[...] = jnp.dot(h, wl_ref[...],
                         preferred_element_type=jnp.float32) + bl_ref[...]


def _fin(p, y, degt, b, wl, bl):
    return pl.pallas_call(
        _fin_body,
        grid=(N // BLK,),
        in_specs=[pl.BlockSpec((NC, BLK, D), lambda i: (0, i, 0)),
                  pl.BlockSpec((BLK, D), lambda i: (i, 0)),
                  pl.BlockSpec((NC, BLK, DEGW), lambda i: (0, i, 0)),
                  pl.BlockSpec((1, D), lambda i: (0, 0)),
                  pl.BlockSpec((D, DOUT), lambda i: (0, 0)),
                  pl.BlockSpec((1, DOUT), lambda i: (0, 0))],
        out_specs=pl.BlockSpec((BLK, DOUT), lambda i: (i, 0)),
        out_shape=jax.ShapeDtypeStruct((N, DOUT), jnp.float32),
    )(p, y, degt, b, wl, bl)


def kernel(x, edge_index, PQVA_mask, target_vector, W1, b1, W2, b2, Wl, bl):
    src = edge_index[0]
    dst = edge_index[1]
    degt = _deg(dst).reshape(NC, N, DEGW)
    xw1 = _mm(x, W1)
    y1 = _scale(xw1, degt)
    p1 = _agg(y1, src, dst).reshape(NC, N, D)
    y2 = _mid(p1, y1, degt, b1.reshape(1, D), W2)
    p2 = _agg(y2, src, dst).reshape(NC, N, D)
    return _fin(p2, y2, degt, b2.reshape(1, D), Wl, bl.reshape(1, DOUT))


# speedup vs baseline: 1.0083x; 1.0073x over previous
import functools

import jax
import jax.numpy as jnp
from jax import lax
from jax.experimental import pallas as pl
from jax.experimental.pallas import tpu as pltpu
from jax.experimental.pallas import tpu_sc as plsc

N = 10000
E = 320000
D = 128
DOUT = 4

NC = 2
NS = 16
NW = NC * NS
CHUNK = 128
NCHUNK = E // CHUNK
BASE_CH = NCHUNK // NW
EXTRA_W = 2
NRING = 4
NGROUP = (BASE_CH + 2 + 3) // 4
RPT = 624
RPT_LAST = N - (NS - 1) * RPT
DEGW = 16

_MESH = plsc.VectorSubcoreMesh(core_axis_name="c", subcore_axis_name="s")


def _fill(ref, rows, width, val):
    vec = jnp.full((16,), val, jnp.float32)

    def body(i, carry):
        for k in range(width // 16):
            ref[i, pl.ds(k * 16, 16)] = vec
        return carry

    lax.fori_loop(0, rows, body, 0)


def _zero_slab(acc, zbuf, row0, nrows):
    full = nrows // CHUNK
    for t in range(full):
        pltpu.sync_copy(zbuf, acc.at[pl.ds(row0 + t * CHUNK, CHUNK)])
    rem = nrows - full * CHUNK
    if rem:
        pltpu.sync_copy(zbuf.at[pl.ds(0, rem)],
                        acc.at[pl.ds(row0 + full * CHUNK, rem)])


def _per_tile_rows(sid, fn):
    @pl.when(sid < NS - 1)
    def _():
        fn(sid * RPT, RPT)

    @pl.when(sid == NS - 1)
    def _():
        fn(sid * RPT, RPT_LAST)


def _worker_span(w):
    start = w * BASE_CH + 2 * jnp.minimum(w, EXTRA_W)
    cnt = BASE_CH + 2 * (w < EXTRA_W).astype(jnp.int32)
    return start, cnt


@functools.partial(
    pl.kernel,
    out_type=jax.ShapeDtypeStruct((NC * N, DEGW), jnp.float32),
    mesh=_MESH,
    scratch_types=[
        pltpu.VMEM_SHARED((N, DEGW), jnp.float32),
        pltpu.VMEM((CHUNK,), jnp.int32),
        pltpu.VMEM((CHUNK,), jnp.int32),
        pltpu.VMEM((CHUNK,), jnp.int32),
        pltpu.VMEM((CHUNK,), jnp.int32),
        pltpu.VMEM((CHUNK,), jnp.int32),
        pltpu.VMEM((CHUNK,), jnp.int32),
        pltpu.VMEM((CHUNK, DEGW), jnp.float32),
        pltpu.VMEM((CHUNK, DEGW), jnp.float32),
        pltpu.SemaphoreType.DMA,
        pltpu.SemaphoreType.DMA,
        pltpu.SemaphoreType.DMA,
        pltpu.SemaphoreType.DMA,
        pltpu.SemaphoreType.DMA,
        pltpu.SemaphoreType.DMA,
    ],
)
def _deg(dst_hbm, out_hbm, acc, dx0, dx1, dx2, dx3, dx4, dx5,
         ones_b, zero_b, i0, i1, i2, i3, i4, i5):
    cid = lax.axis_index("c")
    sid = lax.axis_index("s")
    w = cid * NS + sid
    start, cnt = _worker_span(w)
    didx = (dx0, dx1, dx2, dx3, dx4, dx5)
    isem = (i0, i1, i2, i3, i4, i5)

    _fill(ones_b, CHUNK, DEGW, 1.0)
    _fill(zero_b, CHUNK, DEGW, 0.0)
    _per_tile_rows(sid, lambda r0, nr: _zero_slab(acc, zero_b, r0, nr))
    plsc.subcore_barrier()

    def block(chunk0, n):
        hd = []
        for t in range(n):
            hd.append(pltpu.async_copy(
                dst_hbm.at[pl.ds((chunk0 + t) * CHUNK, CHUNK)],
                didx[t], isem[t]))
        for t in range(n):
            hd[t].wait()
        hc = []
        for t in range(n):
            hc.append(pltpu.async_copy(ones_b, acc.at[didx[t]], isem[t],
                                       add=True))
        for t in range(n):
            hc[t].wait()

    def body(g, carry):
        block(start + 6 * g, 6)
        return carry

    nfull = cnt // 6
    lax.fori_loop(0, nfull, body, 0)

    @pl.when(cnt - 6 * nfull == 2)
    def _():
        block(start + 6 * nfull, 2)
    plsc.subcore_barrier()
    _per_tile_rows(sid, lambda r0, nr: pltpu.sync_copy(
        acc.at[pl.ds(r0, nr)], out_hbm.at[pl.ds(cid * N + r0, nr)]))


AGG_U = 3


@functools.partial(
    pl.kernel,
    out_type=jax.ShapeDtypeStruct((NC * N, D), jnp.float32),
    mesh=_MESH,
    scratch_types=[
        pltpu.VMEM_SHARED((N, D), jnp.float32),
        pltpu.VMEM((CHUNK,), jnp.int32),
        pltpu.VMEM((CHUNK,), jnp.int32),
        pltpu.VMEM((CHUNK,), jnp.int32),
        pltpu.VMEM((CHUNK,), jnp.int32),
        pltpu.VMEM((CHUNK,), jnp.int32),
        pltpu.VMEM((CHUNK,), jnp.int32),
        pltpu.VMEM((CHUNK, D), jnp.float32),
        pltpu.VMEM((CHUNK, D), jnp.float32),
        pltpu.VMEM((CHUNK, D), jnp.float32),
        pltpu.SemaphoreType.DMA,
        pltpu.SemaphoreType.DMA,
        pltpu.SemaphoreType.DMA,
        pltpu.SemaphoreType.DMA,
        pltpu.SemaphoreType.DMA,
        pltpu.SemaphoreType.DMA,
        pltpu.SemaphoreType.DMA,
        pltpu.SemaphoreType.DMA,
        pltpu.SemaphoreType.DMA,
    ],
)
def _agg(y_hbm, src_hbm, dst_hbm, out_hbm, acc,
         sx0, sx1, sx2, dx0, dx1, dx2, rb0, rb1, rb2,
         is0, is1, is2, id0, id1, id2, ig0, ig1, ig2):
    cid = lax.axis_index("c")
    sid = lax.axis_index("s")
    w = cid * NS + sid
    start, cnt = _worker_span(w)
    sidx = (sx0, sx1, sx2)
    didx = (dx0, dx1, dx2)
    rows = (rb0, rb1, rb2)
    isem_s = (is0, is1, is2)
    isem_d = (id0, id1, id2)
    gsem = (ig0, ig1, ig2)

    _fill(rb0, CHUNK, D, 0.0)
    _per_tile_rows(sid, lambda a, n: _zero_slab(acc, rb0, a, n))
    plsc.subcore_barrier()

    def block(chunk0, n):
        hs, hd, hg = [], [], []
        for t in range(n):
            off = (chunk0 + t) * CHUNK
            hs.append(pltpu.async_copy(src_hbm.at[pl.ds(off, CHUNK)],
                                       sidx[t], isem_s[t]))
            hd.append(pltpu.async_copy(dst_hbm.at[pl.ds(off, CHUNK)],
                                       didx[t], isem_d[t]))
        for t in range(n):
            hs[t].wait()
            hg.append(pltpu.async_copy(y_hbm.at[sidx[t]], rows[t], gsem[t]))
        for t in range(n):
            hg[t].wait()
        for t in range(n):
            hd[t].wait()
        hc = []
        for t in range(n):
            hc.append(pltpu.async_copy(rows[t], acc.at[didx[t]], gsem[t],
                                       add=True))
        for t in range(n):
            hc[t].wait()

    def body(g, carry):
        block(start + AGG_U * g, AGG_U)
        return carry

    nfull = cnt // AGG_U
    lax.fori_loop(0, nfull, body, 0)

    @pl.when(cnt - AGG_U * nfull == 2)
    def _():
        block(start + AGG_U * nfull, 2)

    plsc.subcore_barrier()
    _per_tile_rows(sid, lambda a, n: pltpu.sync_copy(
        acc.at[pl.ds(a, n)], out_hbm.at[pl.ds(cid * N + a, n)]))


BLK = 2000


def _dinv_of(degt_ref):
    deg = degt_ref[0, :, 0] + degt_ref[1, :, 0] + 1.0
    return lax.rsqrt(deg)[:, None]


def _mm_body(x_ref, w_ref, o_ref):
    o_ref[...] = jnp.dot(x_ref[...], w_ref[...],
                         preferred_element_type=jnp.float32)


def _mm(x, w):
    m, k = x.shape
    n = w.shape[1]
    return pl.pallas_call(
        _mm_body,
        grid=(m // BLK,),
        in_specs=[pl.BlockSpec((BLK, k), lambda i: (i, 0)),
                  pl.BlockSpec((k, n), lambda i: (0, 0))],
        out_specs=pl.BlockSpec((BLK, n), lambda i: (i, 0)),
        out_shape=jax.ShapeDtypeStruct((m, n), jnp.float32),
    )(x, w)


def _scale_body(xw_ref, degt_ref, y_ref):
    y_ref[...] = xw_ref[...] * _dinv_of(degt_ref)


def _scale(xw, degt):
    return pl.pallas_call(
        _scale_body,
        grid=(N // BLK,),
        in_specs=[pl.BlockSpec((BLK, D), lambda i: (i, 0)),
                  pl.BlockSpec((NC, BLK, DEGW), lambda i: (0, i, 0))],
        out_specs=pl.BlockSpec((BLK, D), lambda i: (i, 0)),
        out_shape=jax.ShapeDtypeStruct((N, D), jnp.float32),
    )(xw, degt)


def _mid_body(p_ref, y_ref, degt_ref, b_ref, w_ref, o_ref):
    dinv = _dinv_of(degt_ref)
    h = (p_ref[0] + p_ref[1] + y_ref[...]) * dinv + b_ref[...]
    h = jnp.maximum(h, 0.0)
    o_ref[...] = jnp.dot(h, w_ref[...],
                         preferred_element_type=jnp.float32) * dinv


def _mid(p, y, degt, b, w):
    return pl.pallas_call(
        _mid_body,
        grid=(N // BLK,),
        in_specs=[pl.BlockSpec((NC, BLK, D), lambda i: (0, i, 0)),
                  pl.BlockSpec((BLK, D), lambda i: (i, 0)),
                  pl.BlockSpec((NC, BLK, DEGW), lambda i: (0, i, 0)),
                  pl.BlockSpec((1, D), lambda i: (0, 0)),
                  pl.BlockSpec((D, D), lambda i: (0, 0))],
        out_specs=pl.BlockSpec((BLK, D), lambda i: (i, 0)),
        out_shape=jax.ShapeDtypeStruct((N, D), jnp.float32),
    )(p, y, degt, b, w)


def _fin_body(p_ref, y_ref, degt_ref, b_ref, wl_ref, bl_ref, o_ref):
    dinv = _dinv_of(degt_ref)
    h = (p_ref[0] + p_ref[1] + y_ref[...]) * dinv + b_ref[...]
    o_ref[...] = jnp.dot(h, wl_ref[...],
                         preferred_element_type=jnp.float32) + bl_ref[...]


def _fin(p, y, degt, b, wl, bl):
    return pl.pallas_call(
        _fin_body,
        grid=(N // BLK,),
        in_specs=[pl.BlockSpec((NC, BLK, D), lambda i: (0, i, 0)),
                  pl.BlockSpec((BLK, D), lambda i: (i, 0)),
                  pl.BlockSpec((NC, BLK, DEGW), lambda i: (0, i, 0)),
                  pl.BlockSpec((1, D), lambda i: (0, 0)),
                  pl.BlockSpec((D, DOUT), lambda i: (0, 0)),
                  pl.BlockSpec((1, DOUT), lambda i: (0, 0))],
        out_specs=pl.BlockSpec((BLK, DOUT), lambda i: (i, 0)),
        out_shape=jax.ShapeDtypeStruct((N, DOUT), jnp.float32),
    )(p, y, degt, b, wl, bl)


def kernel(x, edge_index, PQVA_mask, target_vector, W1, b1, W2, b2, Wl, bl):
    src = edge_index[0]
    dst = edge_index[1]
    degt = _deg(dst).reshape(NC, N, DEGW)
    xw1 = _mm(x, W1)
    y1 = _scale(xw1, degt)
    p1 = _agg(y1, src, dst).reshape(NC, N, D)
    y2 = _mid(p1, y1, degt, b1.reshape(1, D), W2)
    p2 = _agg(y2, src, dst).reshape(NC, N, D)
    return _fin(p2, y2, degt, b2.reshape(1, D), Wl, bl.reshape(1, DOUT))
